# baseline (speedup 1.0000x reference)
import functools

import jax
import jax.numpy as jnp
from jax import lax
from jax.experimental import pallas as pl
from jax.experimental.pallas import tpu as pltpu
from jax.experimental.pallas import tpu_sc as plsc

N = 10000
D = 256
Q = 64
E = 320000
K = 125
NSUB = 16
ER = E // K
CHUNKS = ER // NSUB
NP = 10240
RPS = NP // NSUB


@functools.cache
def _mesh():
    return plsc.VectorSubcoreMesh(core_axis_name="c", subcore_axis_name="s",
                                  num_cores=2, num_subcores=NSUB)


def _sc_agg_body(x4, src_full, dst2, zrows, out, src_v, dst_v, rows_v, agg_sh, sem):
    c = lax.axis_index("c")
    s = lax.axis_index("s")
    pltpu.sync_copy(zrows.at[pl.ds(s * RPS, RPS)],
                    agg_sh.at[pl.ds(s * RPS, RPS)])
    pltpu.sync_copy(src_full.at[pl.ds(c * ER + s * CHUNKS, CHUNKS)], src_v)
    pltpu.sync_copy(dst2.at[pl.ds(s * CHUNKS, CHUNKS)], dst_v)
    plsc.subcore_barrier()

    def chunk(i, carry):
        pltpu.async_copy(x4.at[src_v.at[i]], rows_v, sem).wait()
        pltpu.sync_copy(rows_v, agg_sh.at[dst_v.at[i]], add=True)
        return carry

    lax.fori_loop(0, CHUNKS, chunk, 0)
    plsc.subcore_barrier()
    pltpu.sync_copy(agg_sh.at[pl.ds(s * RPS, RPS)],
                    out.at[pl.ds(c * NP + s * RPS, RPS)])


@functools.cache
def _seg_sum():
    return pl.kernel(
        _sc_agg_body,
        out_type=jax.ShapeDtypeStruct((2 * NP, Q), jnp.float32),
        mesh=_mesh(),
        scratch_types=[
            pltpu.VMEM((CHUNKS, K), jnp.int32),
            pltpu.VMEM((CHUNKS, K), jnp.int32),
            pltpu.VMEM((K, Q), jnp.float32),
            pltpu.VMEM_SHARED((NP, Q), jnp.float32),
            pltpu.SemaphoreType.DMA,
        ],
        compiler_params=pltpu.CompilerParams(use_tc_tiling_on_sc=False),
    )


def _tr_body(xc, emb, wa, wb, b, out):
    x = jnp.dot(xc[...], wa[...], preferred_element_type=jnp.float32)
    x = x + jnp.dot(emb[...], wb[...], preferred_element_type=jnp.float32)
    x = x + b[...]
    for q in range(4):
        out[q * N:(q + 1) * N] = x[:, q * Q:(q + 1) * Q]


_transform = pl.pallas_call(
    _tr_body,
    out_shape=jax.ShapeDtypeStruct((4 * N, Q), jnp.float32),
)


BLK = 2000
NB = N // BLK


def _layer_a_body(p0, p1, inv, x4, lw, lb, rw, new, ps, psq):
    acc = lb[...] + jnp.zeros((BLK, D), jnp.float32)
    for q in range(4):
        aggq = (p0, p1)[q % 2][q // 2] * inv[...]
        acc = acc + jnp.dot(aggq, lw[pl.ds((q // 2) * 2 * Q + (q % 2) * Q, Q), :],
                            preferred_element_type=jnp.float32)
        acc = acc + jnp.dot(x4[q], rw[pl.ds(q * Q, Q), :],
                            preferred_element_type=jnp.float32)
    new[...] = acc
    ps[0] = jnp.sum(acc, axis=0, keepdims=True)
    psq[0] = jnp.sum(acc * acc, axis=0, keepdims=True)


_layer_a = pl.pallas_call(
    _layer_a_body,
    grid=(NB,),
    in_specs=[
        pl.BlockSpec((2, BLK, Q), lambda i: (0, i, 0)),
        pl.BlockSpec((2, BLK, Q), lambda i: (0, i, 0)),
        pl.BlockSpec((BLK, 1), lambda i: (i, 0)),
        pl.BlockSpec((4, BLK, Q), lambda i: (0, i, 0)),
        pl.BlockSpec((D, D), lambda i: (0, 0)),
        pl.BlockSpec((1, D), lambda i: (0, 0)),
        pl.BlockSpec((D, D), lambda i: (0, 0)),
    ],
    out_specs=[
        pl.BlockSpec((BLK, D), lambda i: (i, 0)),
        pl.BlockSpec((1, 1, D), lambda i: (i, 0, 0)),
        pl.BlockSpec((1, 1, D), lambda i: (i, 0, 0)),
    ],
    out_shape=[
        jax.ShapeDtypeStruct((N, D), jnp.float32),
        jax.ShapeDtypeStruct((NB, 1, D), jnp.float32),
        jax.ShapeDtypeStruct((NB, 1, D), jnp.float32),
    ],
)


def _layer_b_body(new, ps, psq, g, b, out):
    m = jnp.sum(ps[:, 0, :], axis=0, keepdims=True) / N
    v = jnp.sum(psq[:, 0, :], axis=0, keepdims=True) / N - m * m
    y = (new[...] - m) / jnp.sqrt(v + 1e-5) * g[...] + b[...]
    y = jnp.where(y >= 0, y, 0.01 * y)
    for q in range(4):
        out[q] = y[:, q * Q:(q + 1) * Q]


_layer_b = pl.pallas_call(
    _layer_b_body,
    grid=(NB,),
    in_specs=[
        pl.BlockSpec((BLK, D), lambda i: (i, 0)),
        pl.BlockSpec((NB, 1, D), lambda i: (0, 0, 0)),
        pl.BlockSpec((NB, 1, D), lambda i: (0, 0, 0)),
        pl.BlockSpec((1, D), lambda i: (0, 0)),
        pl.BlockSpec((1, D), lambda i: (0, 0)),
    ],
    out_specs=[pl.BlockSpec((4, BLK, Q), lambda i: (0, i, 0))],
    out_shape=[jax.ShapeDtypeStruct((4, N, Q), jnp.float32)],
)


def kernel(x_cell, x_town, edge_index_flow, edge_src_spa, edge_dst_spa,
           emb_cell, emb_town, W_tr_cell, b_tr_cell, W_tr_town, b_tr_town,
           flow_lW, flow_lb, flow_rW, spa_lW, spa_lb, spa_rW,
           bn_cell_g, bn_cell_b, bn_town_g, bn_town_b):
    src = edge_index_flow[0]
    dst = edge_index_flow[1]
    src2 = src.reshape(ER, K)
    src_p0 = jnp.concatenate([src2, src2 + 2 * N], axis=0)
    src_p1 = jnp.concatenate([src2 + N, src2 + 3 * N], axis=0)
    dst2 = dst.reshape(ER, K)
    zrows = jnp.zeros((NP, Q), jnp.float32)
    ones4 = jnp.ones((4 * N, Q), jnp.float32)
    zsrc = jnp.zeros((2 * ER, K), jnp.int32)

    x4 = _transform(x_cell, emb_cell[:N], W_tr_cell[:128], W_tr_cell[128:],
                    b_tr_cell.reshape(1, D))
    cnt_full = _seg_sum()(ones4, zsrc, dst2, zrows)
    inv = 1.0 / jnp.maximum(cnt_full[:N, :1], 1.0)
    for l in range(2):
        agg_p0 = _seg_sum()(x4, src_p0, dst2, zrows)
        agg_p1 = _seg_sum()(x4, src_p1, dst2, zrows)
        new, ps, psq = _layer_a(agg_p0.reshape(2, NP, Q), agg_p1.reshape(2, NP, Q),
                                inv, x4.reshape(4, N, Q), flow_lW[l],
                                flow_lb[l].reshape(1, D), flow_rW[l])
        (x4q,) = _layer_b(new, ps, psq, bn_cell_g[l].reshape(1, D),
                          bn_cell_b[l].reshape(1, D))
        x4 = x4q.reshape(4 * N, Q)
    return jnp.concatenate([x4[q * N:(q + 1) * N] for q in range(4)], axis=1)


# speedup vs baseline: 11.9024x; 11.9024x over previous
import functools

import jax
import jax.numpy as jnp
from jax import lax
from jax.experimental import pallas as pl
from jax.experimental.pallas import tpu as pltpu
from jax.experimental.pallas import tpu_sc as plsc

N = 10000
D = 256
Q = 64
E = 320000
K = 125
NSUB = 16
ER = E // K
CHUNKS = ER // NSUB
NP = 10240
RPS = NP // NSUB


@functools.cache
def _mesh():
    return plsc.VectorSubcoreMesh(core_axis_name="c", subcore_axis_name="s",
                                  num_cores=2, num_subcores=NSUB)


def _sc_agg_body(x4, src_full, dst2, zrows, out, src_v, dst_v, rows_v, agg_sh, sem):
    c = lax.axis_index("c")
    s = lax.axis_index("s")
    pltpu.sync_copy(zrows.at[pl.ds(s * RPS, RPS)],
                    agg_sh.at[pl.ds(s * RPS, RPS)])
    pltpu.sync_copy(src_full.at[pl.ds(c * ER + s * CHUNKS, CHUNKS)], src_v)
    pltpu.sync_copy(dst2.at[pl.ds(s * CHUNKS, CHUNKS)], dst_v)
    plsc.subcore_barrier()

    def chunk(i, carry):
        pltpu.async_copy(x4.at[src_v.at[i]], rows_v, sem).wait()
        pltpu.sync_copy(rows_v, agg_sh.at[dst_v.at[i]], add=True)
        return carry

    lax.fori_loop(0, CHUNKS, chunk, 0)
    plsc.subcore_barrier()
    pltpu.sync_copy(agg_sh.at[pl.ds(s * RPS, RPS)],
                    out.at[pl.ds(c * NP + s * RPS, RPS)])


@functools.cache
def _seg_sum():
    return pl.kernel(
        _sc_agg_body,
        out_type=jax.ShapeDtypeStruct((2 * NP, Q), jnp.float32),
        mesh=_mesh(),
        scratch_types=[
            pltpu.VMEM((CHUNKS, K), jnp.int32),
            pltpu.VMEM((CHUNKS, K), jnp.int32),
            pltpu.VMEM((K, Q), jnp.float32),
            pltpu.VMEM_SHARED((NP, Q), jnp.float32),
            pltpu.SemaphoreType.DMA,
        ],
        compiler_params=pltpu.CompilerParams(use_tc_tiling_on_sc=False),
    )


CW = 32
CROWS = ER // CW


def _sc_cnt_body(dst2, z16, ones16, out, dst_v, ones_v, cnt_sh):
    c = lax.axis_index("c")
    s = lax.axis_index("s")
    w = c * NSUB + s
    pltpu.sync_copy(z16.at[pl.ds(s * RPS, RPS)],
                    cnt_sh.at[pl.ds(s * RPS, RPS)])
    pltpu.sync_copy(dst2.at[pl.ds(w * CROWS, CROWS)], dst_v)
    pltpu.sync_copy(ones16, ones_v)
    plsc.subcore_barrier()

    def chunk(i, carry):
        pltpu.sync_copy(ones_v, cnt_sh.at[dst_v.at[i]], add=True)
        return carry

    lax.fori_loop(0, CROWS, chunk, 0)
    plsc.subcore_barrier()
    pltpu.sync_copy(cnt_sh.at[pl.ds(s * RPS, RPS)],
                    out.at[pl.ds(c * NP + s * RPS, RPS)])


@functools.cache
def _counts():
    return pl.kernel(
        _sc_cnt_body,
        out_type=jax.ShapeDtypeStruct((2 * NP, 16), jnp.float32),
        mesh=_mesh(),
        scratch_types=[
            pltpu.VMEM((CROWS, K), jnp.int32),
            pltpu.VMEM((K, 16), jnp.float32),
            pltpu.VMEM_SHARED((NP, 16), jnp.float32),
        ],
        compiler_params=pltpu.CompilerParams(use_tc_tiling_on_sc=False),
    )


def _tr_body(xc, emb, wa, wb, b, out):
    x = jnp.dot(xc[...], wa[...], preferred_element_type=jnp.float32)
    x = x + jnp.dot(emb[...], wb[...], preferred_element_type=jnp.float32)
    x = x + b[...]
    for q in range(4):
        out[q * N:(q + 1) * N] = x[:, q * Q:(q + 1) * Q]


_transform = pl.pallas_call(
    _tr_body,
    out_shape=jax.ShapeDtypeStruct((4 * N, Q), jnp.float32),
)


BLK = 2000
NB = N // BLK


def _layer_a_body(p0, p1, inv, x4, lw, lb, rw, new, ps, psq):
    acc = lb[...] + jnp.zeros((BLK, D), jnp.float32)
    for q in range(4):
        aggq = (p0, p1)[q % 2][q // 2] * inv[...]
        acc = acc + jnp.dot(aggq, lw[pl.ds((q // 2) * 2 * Q + (q % 2) * Q, Q), :],
                            preferred_element_type=jnp.float32)
        acc = acc + jnp.dot(x4[q], rw[pl.ds(q * Q, Q), :],
                            preferred_element_type=jnp.float32)
    new[...] = acc
    ps[0] = jnp.sum(acc, axis=0, keepdims=True)
    psq[0] = jnp.sum(acc * acc, axis=0, keepdims=True)


_layer_a = pl.pallas_call(
    _layer_a_body,
    grid=(NB,),
    in_specs=[
        pl.BlockSpec((2, BLK, Q), lambda i: (0, i, 0)),
        pl.BlockSpec((2, BLK, Q), lambda i: (0, i, 0)),
        pl.BlockSpec((BLK, 1), lambda i: (i, 0)),
        pl.BlockSpec((4, BLK, Q), lambda i: (0, i, 0)),
        pl.BlockSpec((D, D), lambda i: (0, 0)),
        pl.BlockSpec((1, D), lambda i: (0, 0)),
        pl.BlockSpec((D, D), lambda i: (0, 0)),
    ],
    out_specs=[
        pl.BlockSpec((BLK, D), lambda i: (i, 0)),
        pl.BlockSpec((1, 1, D), lambda i: (i, 0, 0)),
        pl.BlockSpec((1, 1, D), lambda i: (i, 0, 0)),
    ],
    out_shape=[
        jax.ShapeDtypeStruct((N, D), jnp.float32),
        jax.ShapeDtypeStruct((NB, 1, D), jnp.float32),
        jax.ShapeDtypeStruct((NB, 1, D), jnp.float32),
    ],
)


def _layer_b_body(new, ps, psq, g, b, out):
    m = jnp.sum(ps[:, 0, :], axis=0, keepdims=True) / N
    v = jnp.sum(psq[:, 0, :], axis=0, keepdims=True) / N - m * m
    y = (new[...] - m) / jnp.sqrt(v + 1e-5) * g[...] + b[...]
    y = jnp.where(y >= 0, y, 0.01 * y)
    for q in range(4):
        out[q] = y[:, q * Q:(q + 1) * Q]


_layer_b = pl.pallas_call(
    _layer_b_body,
    grid=(NB,),
    in_specs=[
        pl.BlockSpec((BLK, D), lambda i: (i, 0)),
        pl.BlockSpec((NB, 1, D), lambda i: (0, 0, 0)),
        pl.BlockSpec((NB, 1, D), lambda i: (0, 0, 0)),
        pl.BlockSpec((1, D), lambda i: (0, 0)),
        pl.BlockSpec((1, D), lambda i: (0, 0)),
    ],
    out_specs=[pl.BlockSpec((4, BLK, Q), lambda i: (0, i, 0))],
    out_shape=[jax.ShapeDtypeStruct((4, N, Q), jnp.float32)],
)


def kernel(x_cell, x_town, edge_index_flow, edge_src_spa, edge_dst_spa,
           emb_cell, emb_town, W_tr_cell, b_tr_cell, W_tr_town, b_tr_town,
           flow_lW, flow_lb, flow_rW, spa_lW, spa_lb, spa_rW,
           bn_cell_g, bn_cell_b, bn_town_g, bn_town_b):
    src = edge_index_flow[0]
    dst = edge_index_flow[1]
    src2 = src.reshape(ER, K)
    src_p0 = jnp.concatenate([src2, src2 + 2 * N], axis=0)
    src_p1 = jnp.concatenate([src2 + N, src2 + 3 * N], axis=0)
    dst2 = dst.reshape(ER, K)
    zrows = jnp.zeros((NP, Q), jnp.float32)
    z16 = jnp.zeros((NP, 16), jnp.float32)
    ones16 = jnp.ones((K, 16), jnp.float32)

    x4 = _transform(x_cell, emb_cell[:N], W_tr_cell[:128], W_tr_cell[128:],
                    b_tr_cell.reshape(1, D))
    cf = _counts()(dst2, z16, ones16)
    inv = 1.0 / jnp.maximum(cf[:N, :1] + cf[NP:NP + N, :1], 1.0)
    for l in range(2):
        agg_p0 = _seg_sum()(x4, src_p0, dst2, zrows)
        agg_p1 = _seg_sum()(x4, src_p1, dst2, zrows)
        new, ps, psq = _layer_a(agg_p0.reshape(2, NP, Q), agg_p1.reshape(2, NP, Q),
                                inv, x4.reshape(4, N, Q), flow_lW[l],
                                flow_lb[l].reshape(1, D), flow_rW[l])
        (x4q,) = _layer_b(new, ps, psq, bn_cell_g[l].reshape(1, D),
                          bn_cell_b[l].reshape(1, D))
        x4 = x4q.reshape(4 * N, Q)
    return jnp.concatenate([x4[q * N:(q + 1) * N] for q in range(4)], axis=1)


# speedup vs baseline: 18.3284x; 1.5399x over previous
import functools

import jax
import jax.numpy as jnp
from jax import lax
from jax.experimental import pallas as pl
from jax.experimental.pallas import tpu as pltpu
from jax.experimental.pallas import tpu_sc as plsc

N = 10000
D = 256
Q = 64
E = 320000
K = 125
NSUB = 16
ER = E // K
CHUNKS = ER // NSUB
NP = 10240
RPS = NP // NSUB


@functools.cache
def _mesh():
    return plsc.VectorSubcoreMesh(core_axis_name="c", subcore_axis_name="s",
                                  num_cores=2, num_subcores=NSUB)


def _sc_agg_body(x4, src_full, dst2, zrows, out, src_v, dst_v,
                 rows0, rows1, agg_sh, gs0, gs1, ss0, ss1):
    c = lax.axis_index("c")
    s = lax.axis_index("s")
    rows = (rows0, rows1)
    gsem = (gs0, gs1)
    ssem = (ss0, ss1)
    pltpu.sync_copy(zrows.at[pl.ds(s * RPS, RPS)],
                    agg_sh.at[pl.ds(s * RPS, RPS)])
    pltpu.sync_copy(src_full.at[pl.ds(c * ER + s * CHUNKS, CHUNKS)], src_v)
    pltpu.sync_copy(dst2.at[pl.ds(s * CHUNKS, CHUNKS)], dst_v)
    plsc.subcore_barrier()

    pltpu.async_copy(x4.at[src_v.at[0]], rows0, gs0)
    pltpu.async_copy(x4.at[src_v.at[1]], rows1, gs1)

    def step(g, carry):
        for b in range(2):
            i = 2 * g + b
            pltpu.make_async_copy(x4.at[src_v.at[0]], rows[b], gsem[b]).wait()
            pltpu.async_copy(rows[b], agg_sh.at[dst_v.at[i]], ssem[b], add=True)
            pltpu.make_async_copy(rows[b], agg_sh.at[dst_v.at[0]], ssem[b]).wait()
            nxt = jnp.minimum(i + 2, CHUNKS - 1)
            pltpu.async_copy(x4.at[src_v.at[nxt]], rows[b], gsem[b])
        return carry

    lax.fori_loop(0, CHUNKS // 2, step, 0)
    pltpu.make_async_copy(x4.at[src_v.at[0]], rows0, gs0).wait()
    pltpu.make_async_copy(x4.at[src_v.at[0]], rows1, gs1).wait()
    plsc.subcore_barrier()
    pltpu.sync_copy(agg_sh.at[pl.ds(s * RPS, RPS)],
                    out.at[pl.ds(c * NP + s * RPS, RPS)])


@functools.cache
def _seg_sum():
    return pl.kernel(
        _sc_agg_body,
        out_type=jax.ShapeDtypeStruct((2 * NP, Q), jnp.float32),
        mesh=_mesh(),
        scratch_types=[
            pltpu.VMEM((CHUNKS, K), jnp.int32),
            pltpu.VMEM((CHUNKS, K), jnp.int32),
            pltpu.VMEM((K, Q), jnp.float32),
            pltpu.VMEM((K, Q), jnp.float32),
            pltpu.VMEM_SHARED((NP, Q), jnp.float32),
            pltpu.SemaphoreType.DMA,
            pltpu.SemaphoreType.DMA,
            pltpu.SemaphoreType.DMA,
            pltpu.SemaphoreType.DMA,
        ],
        compiler_params=pltpu.CompilerParams(use_tc_tiling_on_sc=False),
    )


CW = 32
CROWS = ER // CW


def _sc_cnt_body(dst2, z16, ones16, out, dst_v, ones_v, cnt_sh):
    c = lax.axis_index("c")
    s = lax.axis_index("s")
    w = c * NSUB + s
    pltpu.sync_copy(z16.at[pl.ds(s * RPS, RPS)],
                    cnt_sh.at[pl.ds(s * RPS, RPS)])
    pltpu.sync_copy(dst2.at[pl.ds(w * CROWS, CROWS)], dst_v)
    pltpu.sync_copy(ones16, ones_v)
    plsc.subcore_barrier()

    def chunk(i, carry):
        pltpu.sync_copy(ones_v, cnt_sh.at[dst_v.at[i]], add=True)
        return carry

    lax.fori_loop(0, CROWS, chunk, 0)
    plsc.subcore_barrier()
    pltpu.sync_copy(cnt_sh.at[pl.ds(s * RPS, RPS)],
                    out.at[pl.ds(c * NP + s * RPS, RPS)])


@functools.cache
def _counts():
    return pl.kernel(
        _sc_cnt_body,
        out_type=jax.ShapeDtypeStruct((2 * NP, 16), jnp.float32),
        mesh=_mesh(),
        scratch_types=[
            pltpu.VMEM((CROWS, K), jnp.int32),
            pltpu.VMEM((K, 16), jnp.float32),
            pltpu.VMEM_SHARED((NP, 16), jnp.float32),
        ],
        compiler_params=pltpu.CompilerParams(use_tc_tiling_on_sc=False),
    )


def _tr_body(xc, emb, wa, wb, b, out):
    x = jnp.dot(xc[...], wa[...], preferred_element_type=jnp.float32)
    x = x + jnp.dot(emb[...], wb[...], preferred_element_type=jnp.float32)
    x = x + b[...]
    for q in range(4):
        out[q * N:(q + 1) * N] = x[:, q * Q:(q + 1) * Q]


_transform = pl.pallas_call(
    _tr_body,
    out_shape=jax.ShapeDtypeStruct((4 * N, Q), jnp.float32),
)


BLK = 2000
NB = N // BLK


def _layer_a_body(p0, p1, inv, x4, lw, lb, rw, new, ps, psq):
    acc = lb[...] + jnp.zeros((BLK, D), jnp.float32)
    for q in range(4):
        aggq = (p0, p1)[q % 2][q // 2] * inv[...]
        acc = acc + jnp.dot(aggq, lw[pl.ds((q // 2) * 2 * Q + (q % 2) * Q, Q), :],
                            preferred_element_type=jnp.float32)
        acc = acc + jnp.dot(x4[q], rw[pl.ds(q * Q, Q), :],
                            preferred_element_type=jnp.float32)
    new[...] = acc
    ps[0] = jnp.sum(acc, axis=0, keepdims=True)
    psq[0] = jnp.sum(acc * acc, axis=0, keepdims=True)


_layer_a = pl.pallas_call(
    _layer_a_body,
    grid=(NB,),
    in_specs=[
        pl.BlockSpec((2, BLK, Q), lambda i: (0, i, 0)),
        pl.BlockSpec((2, BLK, Q), lambda i: (0, i, 0)),
        pl.BlockSpec((BLK, 1), lambda i: (i, 0)),
        pl.BlockSpec((4, BLK, Q), lambda i: (0, i, 0)),
        pl.BlockSpec((D, D), lambda i: (0, 0)),
        pl.BlockSpec((1, D), lambda i: (0, 0)),
        pl.BlockSpec((D, D), lambda i: (0, 0)),
    ],
    out_specs=[
        pl.BlockSpec((BLK, D), lambda i: (i, 0)),
        pl.BlockSpec((1, 1, D), lambda i: (i, 0, 0)),
        pl.BlockSpec((1, 1, D), lambda i: (i, 0, 0)),
    ],
    out_shape=[
        jax.ShapeDtypeStruct((N, D), jnp.float32),
        jax.ShapeDtypeStruct((NB, 1, D), jnp.float32),
        jax.ShapeDtypeStruct((NB, 1, D), jnp.float32),
    ],
)


def _layer_b_body(new, ps, psq, g, b, out):
    m = jnp.sum(ps[:, 0, :], axis=0, keepdims=True) / N
    v = jnp.sum(psq[:, 0, :], axis=0, keepdims=True) / N - m * m
    y = (new[...] - m) / jnp.sqrt(v + 1e-5) * g[...] + b[...]
    y = jnp.where(y >= 0, y, 0.01 * y)
    for q in range(4):
        out[q] = y[:, q * Q:(q + 1) * Q]


_layer_b = pl.pallas_call(
    _layer_b_body,
    grid=(NB,),
    in_specs=[
        pl.BlockSpec((BLK, D), lambda i: (i, 0)),
        pl.BlockSpec((NB, 1, D), lambda i: (0, 0, 0)),
        pl.BlockSpec((NB, 1, D), lambda i: (0, 0, 0)),
        pl.BlockSpec((1, D), lambda i: (0, 0)),
        pl.BlockSpec((1, D), lambda i: (0, 0)),
    ],
    out_specs=[pl.BlockSpec((4, BLK, Q), lambda i: (0, i, 0))],
    out_shape=[jax.ShapeDtypeStruct((4, N, Q), jnp.float32)],
)


def kernel(x_cell, x_town, edge_index_flow, edge_src_spa, edge_dst_spa,
           emb_cell, emb_town, W_tr_cell, b_tr_cell, W_tr_town, b_tr_town,
           flow_lW, flow_lb, flow_rW, spa_lW, spa_lb, spa_rW,
           bn_cell_g, bn_cell_b, bn_town_g, bn_town_b):
    src = edge_index_flow[0]
    dst = edge_index_flow[1]
    src2 = src.reshape(ER, K)
    src_p0 = jnp.concatenate([src2, src2 + 2 * N], axis=0)
    src_p1 = jnp.concatenate([src2 + N, src2 + 3 * N], axis=0)
    dst2 = dst.reshape(ER, K)
    zrows = jnp.zeros((NP, Q), jnp.float32)
    z16 = jnp.zeros((NP, 16), jnp.float32)
    ones16 = jnp.ones((K, 16), jnp.float32)

    x4 = _transform(x_cell, emb_cell[:N], W_tr_cell[:128], W_tr_cell[128:],
                    b_tr_cell.reshape(1, D))
    cf = _counts()(dst2, z16, ones16)
    inv = 1.0 / jnp.maximum(cf[:N, :1] + cf[NP:NP + N, :1], 1.0)
    for l in range(2):
        agg_p0 = _seg_sum()(x4, src_p0, dst2, zrows)
        agg_p1 = _seg_sum()(x4, src_p1, dst2, zrows)
        new, ps, psq = _layer_a(agg_p0.reshape(2, NP, Q), agg_p1.reshape(2, NP, Q),
                                inv, x4.reshape(4, N, Q), flow_lW[l],
                                flow_lb[l].reshape(1, D), flow_rW[l])
        (x4q,) = _layer_b(new, ps, psq, bn_cell_g[l].reshape(1, D),
                          bn_cell_b[l].reshape(1, D))
        x4 = x4q.reshape(4 * N, Q)
    return jnp.concatenate([x4[q * N:(q + 1) * N] for q in range(4)], axis=1)


# speedup vs baseline: 18.4599x; 1.0072x over previous
import functools

import jax
import jax.numpy as jnp
from jax import lax
from jax.experimental import pallas as pl
from jax.experimental.pallas import tpu as pltpu
from jax.experimental.pallas import tpu_sc as plsc

N = 10000
D = 256
Q = 64
E = 320000
K = 125
NSUB = 16
ER = E // K
CHUNKS = ER // NSUB
NP = 10240
RPS = NP // NSUB


@functools.cache
def _mesh():
    return plsc.VectorSubcoreMesh(core_axis_name="c", subcore_axis_name="s",
                                  num_cores=2, num_subcores=NSUB)


def _sc_agg_body(x4, src_all, dst2, zrows, out, src_v, dst_v,
                 rows0, rows1, agg_sh, gs0, gs1, ss0, ss1):
    c = lax.axis_index("c")
    s = lax.axis_index("s")
    rows = (rows0, rows1)
    gsem = (gs0, gs1)
    ssem = (ss0, ss1)
    pltpu.sync_copy(zrows.at[pl.ds(s * RPS, RPS)],
                    agg_sh.at[pl.ds(s * RPS, RPS)])
    pltpu.sync_copy(src_all.at[pl.ds(c * ER + s * CHUNKS, CHUNKS)],
                    src_v.at[pl.ds(0, CHUNKS)])
    pltpu.sync_copy(src_all.at[pl.ds(2 * ER + c * ER + s * CHUNKS, CHUNKS)],
                    src_v.at[pl.ds(CHUNKS, CHUNKS)])
    pltpu.sync_copy(dst2.at[pl.ds(s * CHUNKS, CHUNKS)], dst_v)
    plsc.subcore_barrier()

    for p in range(2):
        o = p * CHUNKS
        pltpu.async_copy(x4.at[src_v.at[o]], rows0, gs0)
        pltpu.async_copy(x4.at[src_v.at[o + 1]], rows1, gs1)

        def step(g, carry):
            for b in range(2):
                i = 2 * g + b
                pltpu.make_async_copy(x4.at[src_v.at[0]], rows[b], gsem[b]).wait()
                pltpu.async_copy(rows[b], agg_sh.at[dst_v.at[i]], ssem[b], add=True)
                pltpu.make_async_copy(rows[b], agg_sh.at[dst_v.at[0]], ssem[b]).wait()
                nxt = o + jnp.minimum(i + 2, CHUNKS - 1)
                pltpu.async_copy(x4.at[src_v.at[nxt]], rows[b], gsem[b])
            return carry

        lax.fori_loop(0, CHUNKS // 2, step, 0)
        pltpu.make_async_copy(x4.at[src_v.at[0]], rows0, gs0).wait()
        pltpu.make_async_copy(x4.at[src_v.at[0]], rows1, gs1).wait()
        plsc.subcore_barrier()
        pltpu.sync_copy(agg_sh.at[pl.ds(s * RPS, RPS)],
                        out.at[pl.ds(p * 2 * NP + c * NP + s * RPS, RPS)])
        if p == 0:
            pltpu.sync_copy(zrows.at[pl.ds(s * RPS, RPS)],
                            agg_sh.at[pl.ds(s * RPS, RPS)])
            plsc.subcore_barrier()


@functools.cache
def _seg_sum():
    return pl.kernel(
        _sc_agg_body,
        out_type=jax.ShapeDtypeStruct((4 * NP, Q), jnp.float32),
        mesh=_mesh(),
        scratch_types=[
            pltpu.VMEM((2 * CHUNKS, K), jnp.int32),
            pltpu.VMEM((CHUNKS, K), jnp.int32),
            pltpu.VMEM((K, Q), jnp.float32),
            pltpu.VMEM((K, Q), jnp.float32),
            pltpu.VMEM_SHARED((NP, Q), jnp.float32),
            pltpu.SemaphoreType.DMA,
            pltpu.SemaphoreType.DMA,
            pltpu.SemaphoreType.DMA,
            pltpu.SemaphoreType.DMA,
        ],
        compiler_params=pltpu.CompilerParams(use_tc_tiling_on_sc=False),
    )


CW = 32
CROWS = ER // CW


def _sc_cnt_body(dst2, z16, ones16, out, dst_v, ones_v, cnt_sh):
    c = lax.axis_index("c")
    s = lax.axis_index("s")
    w = c * NSUB + s
    pltpu.sync_copy(z16.at[pl.ds(s * RPS, RPS)],
                    cnt_sh.at[pl.ds(s * RPS, RPS)])
    pltpu.sync_copy(dst2.at[pl.ds(w * CROWS, CROWS)], dst_v)
    pltpu.sync_copy(ones16, ones_v)
    plsc.subcore_barrier()

    def chunk(i, carry):
        pltpu.sync_copy(ones_v, cnt_sh.at[dst_v.at[i]], add=True)
        return carry

    lax.fori_loop(0, CROWS, chunk, 0)
    plsc.subcore_barrier()
    pltpu.sync_copy(cnt_sh.at[pl.ds(s * RPS, RPS)],
                    out.at[pl.ds(c * NP + s * RPS, RPS)])


@functools.cache
def _counts():
    return pl.kernel(
        _sc_cnt_body,
        out_type=jax.ShapeDtypeStruct((2 * NP, 16), jnp.float32),
        mesh=_mesh(),
        scratch_types=[
            pltpu.VMEM((CROWS, K), jnp.int32),
            pltpu.VMEM((K, 16), jnp.float32),
            pltpu.VMEM_SHARED((NP, 16), jnp.float32),
        ],
        compiler_params=pltpu.CompilerParams(use_tc_tiling_on_sc=False),
    )


def _tr_body(xc, emb, wa, wb, b, out):
    x = jnp.dot(xc[...], wa[...], preferred_element_type=jnp.float32)
    x = x + jnp.dot(emb[...], wb[...], preferred_element_type=jnp.float32)
    x = x + b[...]
    for q in range(4):
        out[q * N:(q + 1) * N] = x[:, q * Q:(q + 1) * Q]


_transform = pl.pallas_call(
    _tr_body,
    out_shape=jax.ShapeDtypeStruct((4 * N, Q), jnp.float32),
)


BLK = 2000
NB = N // BLK


def _layer_a_body(agg, inv, x4, lw, lb, rw, new, ps, psq):
    acc = lb[...] + jnp.zeros((BLK, D), jnp.float32)
    for q in range(4):
        aggq = agg[q % 2, q // 2] * inv[...]
        acc = acc + jnp.dot(aggq, lw[pl.ds((q // 2) * 2 * Q + (q % 2) * Q, Q), :],
                            preferred_element_type=jnp.float32)
        acc = acc + jnp.dot(x4[q], rw[pl.ds(q * Q, Q), :],
                            preferred_element_type=jnp.float32)
    new[...] = acc
    ps[0] = jnp.sum(acc, axis=0, keepdims=True)
    psq[0] = jnp.sum(acc * acc, axis=0, keepdims=True)


_layer_a = pl.pallas_call(
    _layer_a_body,
    grid=(NB,),
    in_specs=[
        pl.BlockSpec((2, 2, BLK, Q), lambda i: (0, 0, i, 0)),
        pl.BlockSpec((BLK, 1), lambda i: (i, 0)),
        pl.BlockSpec((4, BLK, Q), lambda i: (0, i, 0)),
        pl.BlockSpec((D, D), lambda i: (0, 0)),
        pl.BlockSpec((1, D), lambda i: (0, 0)),
        pl.BlockSpec((D, D), lambda i: (0, 0)),
    ],
    out_specs=[
        pl.BlockSpec((BLK, D), lambda i: (i, 0)),
        pl.BlockSpec((1, 1, D), lambda i: (i, 0, 0)),
        pl.BlockSpec((1, 1, D), lambda i: (i, 0, 0)),
    ],
    out_shape=[
        jax.ShapeDtypeStruct((N, D), jnp.float32),
        jax.ShapeDtypeStruct((NB, 1, D), jnp.float32),
        jax.ShapeDtypeStruct((NB, 1, D), jnp.float32),
    ],
)


def _layer_b_body(new, ps, psq, g, b, out):
    m = jnp.sum(ps[:, 0, :], axis=0, keepdims=True) / N
    v = jnp.sum(psq[:, 0, :], axis=0, keepdims=True) / N - m * m
    y = (new[...] - m) / jnp.sqrt(v + 1e-5) * g[...] + b[...]
    y = jnp.where(y >= 0, y, 0.01 * y)
    for q in range(4):
        out[q] = y[:, q * Q:(q + 1) * Q]


_layer_b = pl.pallas_call(
    _layer_b_body,
    grid=(NB,),
    in_specs=[
        pl.BlockSpec((BLK, D), lambda i: (i, 0)),
        pl.BlockSpec((NB, 1, D), lambda i: (0, 0, 0)),
        pl.BlockSpec((NB, 1, D), lambda i: (0, 0, 0)),
        pl.BlockSpec((1, D), lambda i: (0, 0)),
        pl.BlockSpec((1, D), lambda i: (0, 0)),
    ],
    out_specs=[pl.BlockSpec((4, BLK, Q), lambda i: (0, i, 0))],
    out_shape=[jax.ShapeDtypeStruct((4, N, Q), jnp.float32)],
)


def kernel(x_cell, x_town, edge_index_flow, edge_src_spa, edge_dst_spa,
           emb_cell, emb_town, W_tr_cell, b_tr_cell, W_tr_town, b_tr_town,
           flow_lW, flow_lb, flow_rW, spa_lW, spa_lb, spa_rW,
           bn_cell_g, bn_cell_b, bn_town_g, bn_town_b):
    src = edge_index_flow[0]
    dst = edge_index_flow[1]
    src2 = src.reshape(ER, K)
    src_all = jnp.concatenate([src2, src2 + 2 * N, src2 + N, src2 + 3 * N],
                              axis=0)
    dst2 = dst.reshape(ER, K)
    zrows = jnp.zeros((NP, Q), jnp.float32)
    z16 = jnp.zeros((NP, 16), jnp.float32)
    ones16 = jnp.ones((K, 16), jnp.float32)

    x4 = _transform(x_cell, emb_cell[:N], W_tr_cell[:128], W_tr_cell[128:],
                    b_tr_cell.reshape(1, D))
    cf = _counts()(dst2, z16, ones16)
    inv = 1.0 / jnp.maximum(cf[:N, :1] + cf[NP:NP + N, :1], 1.0)
    for l in range(2):
        agg = _seg_sum()(x4, src_all, dst2, zrows)
        new, ps, psq = _layer_a(agg.reshape(2, 2, NP, Q),
                                inv, x4.reshape(4, N, Q), flow_lW[l],
                                flow_lb[l].reshape(1, D), flow_rW[l])
        (x4q,) = _layer_b(new, ps, psq, bn_cell_g[l].reshape(1, D),
                          bn_cell_b[l].reshape(1, D))
        x4 = x4q.reshape(4 * N, Q)
    return jnp.concatenate([x4[q * N:(q + 1) * N] for q in range(4)], axis=1)


# speedup vs baseline: 19.8516x; 1.0754x over previous
import functools

import jax
import jax.numpy as jnp
from jax import lax
from jax.experimental import pallas as pl
from jax.experimental.pallas import tpu as pltpu
from jax.experimental.pallas import tpu_sc as plsc

N = 10000
D = 256
Q = 64
E = 320000
K = 125
NSUB = 16
ER = E // K
CHUNKS = ER // NSUB
NP = 10240
RPS = NP // NSUB


@functools.cache
def _mesh():
    return plsc.VectorSubcoreMesh(core_axis_name="c", subcore_axis_name="s",
                                  num_cores=2, num_subcores=NSUB)


def _sc_agg_body(x4, src_all, dst2, zrows, out, src_v, dst_v,
                 rows0, rows1, agg_sh, gs0, gs1, ss0, ss1):
    c = lax.axis_index("c")
    s = lax.axis_index("s")
    rows = (rows0, rows1)
    gsem = (gs0, gs1)
    ssem = (ss0, ss1)
    pltpu.sync_copy(zrows.at[pl.ds(s * RPS, RPS)],
                    agg_sh.at[pl.ds(s * RPS, RPS)])
    pltpu.sync_copy(src_all.at[pl.ds(c * ER + s * CHUNKS, CHUNKS)],
                    src_v.at[pl.ds(0, CHUNKS)])
    pltpu.sync_copy(src_all.at[pl.ds(2 * ER + c * ER + s * CHUNKS, CHUNKS)],
                    src_v.at[pl.ds(CHUNKS, CHUNKS)])
    pltpu.sync_copy(dst2.at[pl.ds(s * CHUNKS, CHUNKS)], dst_v)
    plsc.subcore_barrier()

    for p in range(2):
        o = p * CHUNKS
        pltpu.async_copy(x4.at[src_v.at[o]], rows0, gs0)
        pltpu.async_copy(x4.at[src_v.at[o + 1]], rows1, gs1)

        def step(g, carry):
            for b in range(2):
                i = 2 * g + b
                pltpu.make_async_copy(x4.at[src_v.at[0]], rows[b], gsem[b]).wait()
                pltpu.async_copy(rows[b], agg_sh.at[dst_v.at[i]], ssem[b], add=True)
                pltpu.make_async_copy(rows[b], agg_sh.at[dst_v.at[0]], ssem[b]).wait()
                nxt = o + jnp.minimum(i + 2, CHUNKS - 1)
                pltpu.async_copy(x4.at[src_v.at[nxt]], rows[b], gsem[b])
            return carry

        lax.fori_loop(0, CHUNKS // 2, step, 0)
        pltpu.make_async_copy(x4.at[src_v.at[0]], rows0, gs0).wait()
        pltpu.make_async_copy(x4.at[src_v.at[0]], rows1, gs1).wait()
        plsc.subcore_barrier()
        pltpu.sync_copy(agg_sh.at[pl.ds(s * RPS, RPS)],
                        out.at[pl.ds(p * 2 * NP + c * NP + s * RPS, RPS)])
        if p == 0:
            pltpu.sync_copy(zrows.at[pl.ds(s * RPS, RPS)],
                            agg_sh.at[pl.ds(s * RPS, RPS)])
            plsc.subcore_barrier()


@functools.cache
def _seg_sum():
    return pl.kernel(
        _sc_agg_body,
        out_type=jax.ShapeDtypeStruct((4 * NP, Q), jnp.float32),
        mesh=_mesh(),
        scratch_types=[
            pltpu.VMEM((2 * CHUNKS, K), jnp.int32),
            pltpu.VMEM((CHUNKS, K), jnp.int32),
            pltpu.VMEM((K, Q), jnp.float32),
            pltpu.VMEM((K, Q), jnp.float32),
            pltpu.VMEM_SHARED((NP, Q), jnp.float32),
            pltpu.SemaphoreType.DMA,
            pltpu.SemaphoreType.DMA,
            pltpu.SemaphoreType.DMA,
            pltpu.SemaphoreType.DMA,
        ],
        compiler_params=pltpu.CompilerParams(use_tc_tiling_on_sc=False),
    )


CW = 32
CROWS = ER // CW


def _sc_cnt_body(dst2, z16, ones16, out, dst_v, ones_v, cnt_sh):
    c = lax.axis_index("c")
    s = lax.axis_index("s")
    w = c * NSUB + s
    pltpu.sync_copy(z16.at[pl.ds(s * RPS, RPS)],
                    cnt_sh.at[pl.ds(s * RPS, RPS)])
    pltpu.sync_copy(dst2.at[pl.ds(w * CROWS, CROWS)], dst_v)
    pltpu.sync_copy(ones16, ones_v)
    plsc.subcore_barrier()

    def chunk(i, carry):
        pltpu.sync_copy(ones_v, cnt_sh.at[dst_v.at[i]], add=True)
        return carry

    lax.fori_loop(0, CROWS, chunk, 0)
    plsc.subcore_barrier()
    pltpu.sync_copy(cnt_sh.at[pl.ds(s * RPS, RPS)],
                    out.at[pl.ds(c * NP + s * RPS, RPS)])


@functools.cache
def _counts():
    return pl.kernel(
        _sc_cnt_body,
        out_type=jax.ShapeDtypeStruct((2 * NP, 16), jnp.float32),
        mesh=_mesh(),
        scratch_types=[
            pltpu.VMEM((CROWS, K), jnp.int32),
            pltpu.VMEM((K, 16), jnp.float32),
            pltpu.VMEM_SHARED((NP, 16), jnp.float32),
        ],
        compiler_params=pltpu.CompilerParams(use_tc_tiling_on_sc=False),
    )


def _tr_body(xc, emb, wa, wb, b, out):
    x = jnp.dot(xc[...], wa[...], preferred_element_type=jnp.float32)
    x = x + jnp.dot(emb[...], wb[...], preferred_element_type=jnp.float32)
    x = x + b[...]
    for q in range(4):
        out[q * N:(q + 1) * N] = x[:, q * Q:(q + 1) * Q]


_transform = pl.pallas_call(
    _tr_body,
    out_shape=jax.ShapeDtypeStruct((4 * N, Q), jnp.float32),
)


BLK = 2000
NB = N // BLK


def _layer_fused_body(agg, inv, x4, lw, lb, rw, g, b, out, new_buf, sacc, *, last):
    ph = pl.program_id(0)
    i = pl.program_id(1)

    @pl.when(ph == 0)
    def _compute():
        acc = lb[...] + jnp.zeros((BLK, D), jnp.float32)
        for q in range(4):
            aggq = agg[q % 2, q // 2] * inv[...]
            acc2 = jnp.dot(aggq, lw[pl.ds((q // 2) * 2 * Q + (q % 2) * Q, Q), :],
                           preferred_element_type=jnp.float32)
            acc3 = jnp.dot(x4[q], rw[pl.ds(q * Q, Q), :],
                           preferred_element_type=jnp.float32)
            acc = acc + acc2 + acc3
        new_buf[pl.ds(i * BLK, BLK), :] = acc
        ps = jnp.sum(acc, axis=0, keepdims=True)
        psq = jnp.sum(acc * acc, axis=0, keepdims=True)

        @pl.when(i == 0)
        def _():
            sacc[0:1] = ps
            sacc[1:2] = psq

        @pl.when(i > 0)
        def _():
            sacc[0:1] += ps
            sacc[1:2] += psq

    @pl.when(ph == 1)
    def _apply():
        m = sacc[0:1] / N
        v = sacc[1:2] / N - m * m
        y = (new_buf[pl.ds(i * BLK, BLK), :] - m) / jnp.sqrt(v + 1e-5) * g[...] + b[...]
        y = jnp.where(y >= 0, y, 0.01 * y)
        if last:
            out[...] = y
        else:
            for q in range(4):
                out[q] = y[:, q * Q:(q + 1) * Q]


def _mk_layer(last):
    return pl.pallas_call(
        functools.partial(_layer_fused_body, last=last),
        grid=(2, NB),
        in_specs=[
            pl.BlockSpec((2, 2, BLK, Q), lambda ph, i: (0, 0, i * (1 - ph), 0)),
            pl.BlockSpec((BLK, 1), lambda ph, i: (i * (1 - ph), 0)),
            pl.BlockSpec((4, BLK, Q), lambda ph, i: (0, i * (1 - ph), 0)),
            pl.BlockSpec((D, D), lambda ph, i: (0, 0)),
            pl.BlockSpec((1, D), lambda ph, i: (0, 0)),
            pl.BlockSpec((D, D), lambda ph, i: (0, 0)),
            pl.BlockSpec((1, D), lambda ph, i: (0, 0)),
            pl.BlockSpec((1, D), lambda ph, i: (0, 0)),
        ],
        out_specs=(pl.BlockSpec((BLK, D), lambda ph, i: (i * ph, 0)) if last
                   else pl.BlockSpec((4, BLK, Q), lambda ph, i: (0, i * ph, 0))),
        out_shape=(jax.ShapeDtypeStruct((N, D), jnp.float32) if last
                   else jax.ShapeDtypeStruct((4, N, Q), jnp.float32)),
        scratch_shapes=[
            pltpu.VMEM((N, D), jnp.float32),
            pltpu.VMEM((8, D), jnp.float32),
        ],
    )


_layer_mid = _mk_layer(False)
_layer_last = _mk_layer(True)


def kernel(x_cell, x_town, edge_index_flow, edge_src_spa, edge_dst_spa,
           emb_cell, emb_town, W_tr_cell, b_tr_cell, W_tr_town, b_tr_town,
           flow_lW, flow_lb, flow_rW, spa_lW, spa_lb, spa_rW,
           bn_cell_g, bn_cell_b, bn_town_g, bn_town_b):
    src = edge_index_flow[0]
    dst = edge_index_flow[1]
    src2 = src.reshape(ER, K)
    src_all = jnp.concatenate([src2, src2 + 2 * N, src2 + N, src2 + 3 * N],
                              axis=0)
    dst2 = dst.reshape(ER, K)
    zrows = jnp.zeros((NP, Q), jnp.float32)
    z16 = jnp.zeros((NP, 16), jnp.float32)
    ones16 = jnp.ones((K, 16), jnp.float32)

    x4 = _transform(x_cell, emb_cell[:N], W_tr_cell[:128], W_tr_cell[128:],
                    b_tr_cell.reshape(1, D))
    cf = _counts()(dst2, z16, ones16)
    inv = 1.0 / jnp.maximum(cf[:N, :1] + cf[NP:NP + N, :1], 1.0)
    for l in range(2):
        agg = _seg_sum()(x4, src_all, dst2, zrows)
        step = _layer_last if l == 1 else _layer_mid
        x4 = step(agg.reshape(2, 2, NP, Q), inv, x4.reshape(4, N, Q),
                  flow_lW[l], flow_lb[l].reshape(1, D), flow_rW[l],
                  bn_cell_g[l].reshape(1, D), bn_cell_b[l].reshape(1, D))
        if l == 0:
            x4 = x4.reshape(4 * N, Q)
    return x4
